# baseline (device time: 305747 ns/iter reference)
import jax
import jax.numpy as jnp
from jax import lax
from jax.experimental import pallas as pl
from jax.experimental.pallas import tpu as pltpu

N_DEV = 4


def kernel(x, w_mat, scale_x, scale_w):
    m, k_shard = x.shape
    _, n = w_mat.shape
    m_chunk = m // N_DEV
    n_half = n // 2
    sub = n_half // 2
    n_hops = N_DEV - 1

    def body(x_ref, w_ref, sx_ref, sw_ref, out_ref,
             wb_ref, accR_ref, accL_ref, commR_ref, commL_ref,
             send_sems, recv_sems, copy_sems):
        my = lax.axis_index("i")
        left = (my + N_DEV - 1) % N_DEV
        right = (my + 1) % N_DEV

        barrier_sem = pltpu.get_barrier_semaphore()
        for nbr in (left, right):
            pl.semaphore_signal(
                barrier_sem, inc=1,
                device_id=(nbr,), device_id_type=pl.DeviceIdType.MESH,
            )
        pl.semaphore_wait(barrier_sem, 2)

        wb_ref[...] = w_ref[...].astype(jnp.bfloat16)

        acc = (accR_ref, accL_ref)
        comm = (commR_ref, commL_ref)
        col0 = (0, n_half)
        dev = (right, left)

        def gemm(c, buf, d):
            xb = x_ref[pl.ds(c * m_chunk, m_chunk), :].astype(jnp.bfloat16)
            acc[d][buf] = lax.dot_general(
                xb, wb_ref[:, col0[d]:col0[d] + n_half],
                (((1,), (0,)), ((), ())),
                preferred_element_type=jnp.float32,
            )

        def rs_rdma(s, h, d):
            cs = slice(h * sub, (h + 1) * sub)
            return pltpu.make_async_remote_copy(
                src_ref=acc[d].at[s % 2, :, cs],
                dst_ref=comm[d].at[s, :, cs],
                send_sem=send_sems.at[d, s, h],
                recv_sem=recv_sems.at[d, s, h],
                device_id=(dev[d],), device_id_type=pl.DeviceIdType.MESH,
            )

        def ag_rdma(t, h, d):
            cR = (my + 1 - t) % N_DEV if d == 0 else (my + N_DEV - 1 + t) % N_DEV
            dst = out_ref.at[pl.ds(cR * m_chunk, m_chunk),
                             pl.ds(col0[d] + h * sub, sub)]
            src = acc[d].at[n_hops % 2, :, h * sub:(h + 1) * sub] if t == 0 else dst
            return pltpu.make_async_remote_copy(
                src_ref=src, dst_ref=dst,
                send_sem=send_sems.at[d, n_hops + t, h],
                recv_sem=recv_sems.at[d, n_hops + t, h],
                device_id=(dev[d],), device_id_type=pl.DeviceIdType.MESH,
            )

        scale = sx_ref[0] * sw_ref[0]
        fb = n_hops % 2

        gemm(my, 0, 0)
        gemm(my, 0, 1)
        for d in (0, 1):
            for h in (0, 1):
                rs_rdma(0, h, d).start()

        for s in range(n_hops):
            wbuf = (s + 1) % 2
            if s >= 1:
                for d in (0, 1):
                    for h in (0, 1):
                        rs_rdma(s - 1, h, d).wait_send()
            gemm((my - s - 1) % N_DEV, wbuf, 0)
            gemm((my + s + 1) % N_DEV, wbuf, 1)
            for h in (0, 1):
                for d in (0, 1):
                    cs = slice(h * sub, (h + 1) * sub)
                    rs_rdma(s, h, d).wait_recv()
                    acc[d][wbuf, :, cs] = (
                        acc[d][wbuf, :, cs] + comm[d][s, :, cs]
                    )
                    if s < n_hops - 1:
                        rs_rdma(s + 1, h, d).start()
                    else:
                        y = acc[d][fb, :, cs] * scale
                        acc[d][fb, :, cs] = y / (
                            1.0 + jnp.exp(-jnp.clip(y, -60.0, 60.0))
                        )
                        ag_rdma(0, h, d).start()

        own_c = ((my + 1) % N_DEV, (my + N_DEV - 1) % N_DEV)
        cps = []
        for d in (0, 1):
            cp = pltpu.make_async_copy(
                acc[d].at[fb],
                out_ref.at[pl.ds(own_c[d] * m_chunk, m_chunk),
                           pl.ds(col0[d], n_half)],
                copy_sems.at[d],
            )
            cp.start()
            cps.append(cp)

        for t in range(n_hops):
            for h in (0, 1):
                for d in (0, 1):
                    ag_rdma(t, h, d).wait_recv()
                    if t < n_hops - 1:
                        ag_rdma(t + 1, h, d).start()

        for d in (0, 1):
            for h in (0, 1):
                rs_rdma(n_hops - 1, h, d).wait_send()
                for t in range(n_hops):
                    ag_rdma(t, h, d).wait_send()
        for cp in cps:
            cp.wait()

    return pl.pallas_call(
        body,
        out_shape=jax.ShapeDtypeStruct((m, n), jnp.float32),
        in_specs=[
            pl.BlockSpec(memory_space=pltpu.VMEM),
            pl.BlockSpec(memory_space=pltpu.VMEM),
            pl.BlockSpec(memory_space=pltpu.SMEM),
            pl.BlockSpec(memory_space=pltpu.SMEM),
        ],
        out_specs=pl.BlockSpec(memory_space=pl.ANY),
        scratch_shapes=[
            pltpu.VMEM((k_shard, n), jnp.bfloat16),
            pltpu.VMEM((2, m_chunk, n_half), jnp.float32),
            pltpu.VMEM((2, m_chunk, n_half), jnp.float32),
            pltpu.VMEM((n_hops, m_chunk, n_half), jnp.float32),
            pltpu.VMEM((n_hops, m_chunk, n_half), jnp.float32),
            pltpu.SemaphoreType.DMA((2, 2 * n_hops, 2)),
            pltpu.SemaphoreType.DMA((2, 2 * n_hops, 2)),
            pltpu.SemaphoreType.DMA((2,)),
        ],
        compiler_params=pltpu.CompilerParams(
            collective_id=0,
            vmem_limit_bytes=56 * 1024 * 1024,
        ),
    )(x, w_mat, scale_x, scale_w)


# device time: 303582 ns/iter; 1.0071x vs baseline; 1.0071x over previous
import jax
import jax.numpy as jnp
from jax import lax
from jax.experimental import pallas as pl
from jax.experimental.pallas import tpu as pltpu

N_DEV = 4


def kernel(x, w_mat, scale_x, scale_w):
    m, k_shard = x.shape
    _, n = w_mat.shape
    m_chunk = m // N_DEV
    n_half = n // 2
    sub = n_half // 2
    n_hops = N_DEV - 1

    def body(x_ref, w_ref, sx_ref, sw_ref, out_ref,
             accR_ref, accL_ref, finR_ref, finL_ref,
             commR_ref, commL_ref, send_sems, recv_sems, copy_sems):
        my = lax.axis_index("i")
        left = (my + N_DEV - 1) % N_DEV
        right = (my + 1) % N_DEV

        barrier_sem = pltpu.get_barrier_semaphore()
        for nbr in (left, right):
            pl.semaphore_signal(
                barrier_sem, inc=1,
                device_id=(nbr,), device_id_type=pl.DeviceIdType.MESH,
            )
        pl.semaphore_wait(barrier_sem, 2)

        acc = (accR_ref, accL_ref)
        fin = (finR_ref, finL_ref)
        comm = (commR_ref, commL_ref)
        col0 = (0, n_half)
        dev = (right, left)

        def gemm(c, buf, d, h=None):
            lo = col0[d] if h is None else col0[d] + h * sub
            width = n_half if h is None else sub
            res = lax.dot_general(
                x_ref[pl.ds(c * m_chunk, m_chunk), :],
                w_ref[:, lo:lo + width],
                (((1,), (0,)), ((), ())),
                preferred_element_type=jnp.int32,
            )
            if h is None:
                acc[d][buf] = res
            else:
                acc[d][buf, :, h * sub:(h + 1) * sub] = res

        def rs_rdma(s, h, d):
            cs = slice(h * sub, (h + 1) * sub)
            return pltpu.make_async_remote_copy(
                src_ref=acc[d].at[s % 2, :, cs],
                dst_ref=comm[d].at[s, :, cs],
                send_sem=send_sems.at[d, s, h],
                recv_sem=recv_sems.at[d, s, h],
                device_id=(dev[d],), device_id_type=pl.DeviceIdType.MESH,
            )

        def ag_rdma(t, h, d):
            c = (my + 1 - t) % N_DEV if d == 0 else (my + N_DEV - 1 + t) % N_DEV
            dst = out_ref.at[pl.ds(c * m_chunk, m_chunk),
                             pl.ds(col0[d] + h * sub, sub)]
            src = fin[d].at[:, h * sub:(h + 1) * sub] if t == 0 else dst
            return pltpu.make_async_remote_copy(
                src_ref=src, dst_ref=dst,
                send_sem=send_sems.at[d, n_hops + t, h],
                recv_sem=recv_sems.at[d, n_hops + t, h],
                device_id=(dev[d],), device_id_type=pl.DeviceIdType.MESH,
            )

        scale = sx_ref[0] * sw_ref[0]
        fb = n_hops % 2

        for h in (0, 1):
            for d in (0, 1):
                gemm(my, 0, d, h)
                rs_rdma(0, h, d).start()

        for s in range(n_hops):
            wbuf = (s + 1) % 2
            if s >= 1:
                for d in (0, 1):
                    for h in (0, 1):
                        rs_rdma(s - 1, h, d).wait_send()
            gemm((my - s - 1) % N_DEV, wbuf, 0)
            gemm((my + s + 1) % N_DEV, wbuf, 1)
            for h in (0, 1):
                for d in (0, 1):
                    cs = slice(h * sub, (h + 1) * sub)
                    rs_rdma(s, h, d).wait_recv()
                    acc[d][wbuf, :, cs] = (
                        acc[d][wbuf, :, cs] + comm[d][s, :, cs]
                    )
                    if s < n_hops - 1:
                        rs_rdma(s + 1, h, d).start()
                    else:
                        y = acc[d][fb, :, cs].astype(jnp.float32) * scale
                        fin[d][:, cs] = y / (
                            1.0 + jnp.exp(-jnp.clip(y, -60.0, 60.0))
                        )
                        ag_rdma(0, h, d).start()

        own_c = ((my + 1) % N_DEV, (my + N_DEV - 1) % N_DEV)
        cps = []
        for d in (0, 1):
            cp = pltpu.make_async_copy(
                fin[d],
                out_ref.at[pl.ds(own_c[d] * m_chunk, m_chunk),
                           pl.ds(col0[d], n_half)],
                copy_sems.at[d],
            )
            cp.start()
            cps.append(cp)

        for t in range(n_hops):
            for h in (0, 1):
                for d in (0, 1):
                    ag_rdma(t, h, d).wait_recv()
                    if t < n_hops - 1:
                        ag_rdma(t + 1, h, d).start()

        for d in (0, 1):
            for h in (0, 1):
                rs_rdma(n_hops - 1, h, d).wait_send()
                for t in range(n_hops):
                    ag_rdma(t, h, d).wait_send()
        for cp in cps:
            cp.wait()

    return pl.pallas_call(
        body,
        out_shape=jax.ShapeDtypeStruct((m, n), jnp.float32),
        in_specs=[
            pl.BlockSpec(memory_space=pltpu.VMEM),
            pl.BlockSpec(memory_space=pltpu.VMEM),
            pl.BlockSpec(memory_space=pltpu.SMEM),
            pl.BlockSpec(memory_space=pltpu.SMEM),
        ],
        out_specs=pl.BlockSpec(memory_space=pl.ANY),
        scratch_shapes=[
            pltpu.VMEM((2, m_chunk, n_half), jnp.int32),
            pltpu.VMEM((2, m_chunk, n_half), jnp.int32),
            pltpu.VMEM((m_chunk, n_half), jnp.float32),
            pltpu.VMEM((m_chunk, n_half), jnp.float32),
            pltpu.VMEM((n_hops, m_chunk, n_half), jnp.int32),
            pltpu.VMEM((n_hops, m_chunk, n_half), jnp.int32),
            pltpu.SemaphoreType.DMA((2, 2 * n_hops, 2)),
            pltpu.SemaphoreType.DMA((2, 2 * n_hops, 2)),
            pltpu.SemaphoreType.DMA((2,)),
        ],
        compiler_params=pltpu.CompilerParams(
            collective_id=0,
            vmem_limit_bytes=56 * 1024 * 1024,
        ),
    )(x, w_mat, scale_x, scale_w)
